# Initial kernel scaffold; baseline (speedup 1.0000x reference)
import functools

import jax
import jax.numpy as jnp
from jax import lax
from jax.experimental import pallas as pl
from jax.experimental.pallas import tpu as pltpu
from jax.experimental.pallas import tpu_sc as plsc

B = 16384
LSEQ = 200
D = 16
OUT = 2

NC = 2
NS = 16
NW = NC * NS

ROWS_PER_W = B // NW
CHUNK = 16
NCHUNK = ROWS_PER_W // CHUNK
IDX_PER_CHUNK = CHUNK * LSEQ
GK = 160
NG = IDX_PER_CHUNK // GK
NACC = 8


def _sc_emb_pool_sum(x_flat, emb_table):
    mesh = plsc.VectorSubcoreMesh(core_axis_name="c", subcore_axis_name="s")

    @functools.partial(
        pl.kernel,
        mesh=mesh,
        out_type=jax.ShapeDtypeStruct((B, D), jnp.float32),
        scratch_types=[
            pltpu.VMEM((IDX_PER_CHUNK,), jnp.int32),
            pltpu.VMEM((IDX_PER_CHUNK, D), jnp.float32),
            pltpu.VMEM((CHUNK, D), jnp.float32),
            pltpu.SemaphoreType.DMA,
        ],
    )
    def emb_pool(x_hbm, table_hbm, out_hbm, idx_v, rows_v, pool_v, sem):
        wid = lax.axis_index("s") * NC + lax.axis_index("c")

        def chunk_body(c, carry):
            row0 = wid * ROWS_PER_W + c * CHUNK
            flat0 = pl.multiple_of(row0 * LSEQ, IDX_PER_CHUNK)
            pltpu.sync_copy(x_hbm.at[pl.ds(flat0, IDX_PER_CHUNK)], idx_v)
            copies = [
                pltpu.async_copy(
                    table_hbm.at[idx_v.at[pl.ds(j * GK, GK)]],
                    rows_v.at[pl.ds(j * GK, GK)],
                    sem,
                )
                for j in range(NG)
            ]
            for cp in copies:
                cp.wait()

            def row_body(r, rcarry):
                base = r * LSEQ
                accs = [rows_v[base + k, :] for k in range(NACC)]
                for j in range(NACC, LSEQ):
                    accs[j % NACC] = accs[j % NACC] + rows_v[base + j, :]
                s = accs[0]
                for a in accs[1:]:
                    s = s + a
                pool_v[r, :] = s
                return rcarry

            lax.fori_loop(0, CHUNK, row_body, 0)
            pltpu.sync_copy(pool_v, out_hbm.at[pl.ds(row0, CHUNK)])
            return carry

        lax.fori_loop(0, NCHUNK, chunk_body, 0)

    return emb_pool(x_flat, emb_table)


def _mlp_body(s_ref, w1_ref, b1_ref, w2_ref, b2_ref, o_ref):
    s = s_ref[...]
    h = jnp.dot(s, w1_ref[...], preferred_element_type=jnp.float32)
    h = jnp.maximum(h + b1_ref[...], 0.0)
    o = jnp.dot(h, w2_ref[...], preferred_element_type=jnp.float32)
    o_ref[...] = o + b2_ref[...]


def _tc_mlp(sums, w1t, b1, w2t, b2):
    return pl.pallas_call(
        _mlp_body,
        out_shape=jax.ShapeDtypeStruct((B, OUT), jnp.float32),
    )(sums, w1t, b1, w2t, b2)


def kernel(x, emb_table, W1, b1, W2, b2):
    x_flat = x.reshape(B * LSEQ)
    sums = _sc_emb_pool_sum(x_flat, emb_table)
    w1t = W1.T / float(LSEQ)
    w2t = W2.T
    return _tc_mlp(sums, w1t, b1.reshape(1, D), w2t, b2.reshape(1, OUT))


# speedup vs baseline: 7.9114x; 7.9114x over previous
import functools

import jax
import jax.numpy as jnp
from jax import lax
from jax.experimental import pallas as pl
from jax.experimental.pallas import tpu as pltpu
from jax.experimental.pallas import tpu_sc as plsc

B = 16384
LSEQ = 200
D = 16
OUT = 2

NC = 2
NS = 16
NW = NC * NS

ROWS_PER_W = B // NW
CHUNK = 16
NCHUNK = ROWS_PER_W // CHUNK
IDX_PER_CHUNK = CHUNK * LSEQ
GK = 160
NG = IDX_PER_CHUNK // GK
NACC = 8


def _sc_emb_pool_sum(x_flat, emb_table):
    mesh = plsc.VectorSubcoreMesh(core_axis_name="c", subcore_axis_name="s")

    @functools.partial(
        pl.kernel,
        mesh=mesh,
        out_type=jax.ShapeDtypeStruct((B, D), jnp.float32),
        scratch_types=[
            pltpu.VMEM((IDX_PER_CHUNK,), jnp.int32),
            pltpu.VMEM((IDX_PER_CHUNK, D), jnp.float32),
            pltpu.VMEM((CHUNK, D), jnp.float32),
            pltpu.SemaphoreType.DMA,
        ],
        compiler_params=pltpu.CompilerParams(use_tc_tiling_on_sc=False),
    )
    def emb_pool(x_hbm, table_hbm, out_hbm, idx_v, rows_v, pool_v, sem):
        wid = lax.axis_index("s") * NC + lax.axis_index("c")

        def chunk_body(c, carry):
            row0 = wid * ROWS_PER_W + c * CHUNK
            flat0 = pl.multiple_of(row0 * LSEQ, IDX_PER_CHUNK)
            pltpu.sync_copy(x_hbm.at[pl.ds(flat0, IDX_PER_CHUNK)], idx_v)
            copies = [
                pltpu.async_copy(
                    table_hbm.at[idx_v.at[pl.ds(j * GK, GK)]],
                    rows_v.at[pl.ds(j * GK, GK)],
                    sem,
                )
                for j in range(NG)
            ]
            for cp in copies:
                cp.wait()

            def row_body(r, rcarry):
                base = r * LSEQ
                accs = [rows_v[base + k, :] for k in range(NACC)]
                for j in range(NACC, LSEQ):
                    accs[j % NACC] = accs[j % NACC] + rows_v[base + j, :]
                s = accs[0]
                for a in accs[1:]:
                    s = s + a
                pool_v[r, :] = s
                return rcarry

            lax.fori_loop(0, CHUNK, row_body, 0)
            pltpu.sync_copy(pool_v, out_hbm.at[pl.ds(row0, CHUNK)])
            return carry

        lax.fori_loop(0, NCHUNK, chunk_body, 0)

    return emb_pool(x_flat, emb_table)


def _mlp_body(s_ref, w1_ref, b1_ref, w2_ref, b2_ref, o_ref):
    s = s_ref[...]
    h = jnp.dot(s, w1_ref[...], preferred_element_type=jnp.float32)
    h = jnp.maximum(h + b1_ref[...], 0.0)
    o = jnp.dot(h, w2_ref[...], preferred_element_type=jnp.float32)
    o_ref[...] = o + b2_ref[...]


def _tc_mlp(sums, w1t, b1, w2t, b2):
    return pl.pallas_call(
        _mlp_body,
        out_shape=jax.ShapeDtypeStruct((B, OUT), jnp.float32),
    )(sums, w1t, b1, w2t, b2)


def kernel(x, emb_table, W1, b1, W2, b2):
    x_flat = x.reshape(B * LSEQ)
    sums = _sc_emb_pool_sum(x_flat, emb_table)
    w1t = W1.T / float(LSEQ)
    w2t = W2.T
    return _tc_mlp(sums, w1t, b1.reshape(1, D), w2t, b2.reshape(1, OUT))


# speedup vs baseline: 8.8916x; 1.1239x over previous
import functools

import jax
import jax.numpy as jnp
from jax import lax
from jax.experimental import pallas as pl
from jax.experimental.pallas import tpu as pltpu
from jax.experimental.pallas import tpu_sc as plsc

B = 16384
LSEQ = 200
D = 16
OUT = 2

NC = 2
NS = 16
NW = NC * NS

ROWS_PER_W = B // NW
CHUNK = 16
SUPER = 2 * CHUNK
NSUPER = ROWS_PER_W // SUPER
NACC = 8
ROWS_BUF = CHUNK * LSEQ


def _sc_emb_pool_sum(x, emb_table):
    mesh = plsc.VectorSubcoreMesh(core_axis_name="c", subcore_axis_name="s")

    @functools.partial(
        pl.kernel,
        mesh=mesh,
        out_type=jax.ShapeDtypeStruct((B, D), jnp.float32),
        scratch_types=[
            pltpu.VMEM((SUPER, LSEQ), jnp.int32),
            pltpu.VMEM((SUPER, LSEQ), jnp.int32),
            pltpu.VMEM((ROWS_BUF, D), jnp.float32),
            pltpu.VMEM((ROWS_BUF, D), jnp.float32),
            pltpu.VMEM((4, CHUNK, D), jnp.float32),
            pltpu.SemaphoreType.DMA,
            pltpu.SemaphoreType.DMA,
            pltpu.SemaphoreType.DMA,
            pltpu.SemaphoreType.DMA,
        ],
        compiler_params=pltpu.CompilerParams(use_tc_tiling_on_sc=False),
    )
    def emb_pool(x_hbm, table_hbm, out_hbm, idx_a, idx_b, rows0, rows1,
                 pool_v, sem0, sem1, sem_i, sem_p):
        wid = lax.axis_index("s") * NC + lax.axis_index("c")
        wrow0 = wid * ROWS_PER_W

        def fire_gathers(idx_ref, roff, rows_ref, sem):
            for r in range(CHUNK):
                pltpu.async_copy(
                    table_hbm.at[idx_ref.at[roff + r]],
                    rows_ref.at[pl.ds(r * LSEQ, LSEQ)],
                    sem,
                )

        def drain_rows(rows_ref, sem):
            pltpu.make_async_copy(
                table_hbm.at[pl.ds(0, ROWS_BUF)], rows_ref, sem).wait()

        def drain_idx(idx_ref):
            pltpu.make_async_copy(
                x_hbm.at[pl.ds(0, SUPER)], idx_ref, sem_i).wait()

        def drain_pools(n):
            for _ in range(n):
                pltpu.make_async_copy(
                    pool_v.at[0], out_hbm.at[pl.ds(0, CHUNK)], sem_p).wait()

        def reduce_store(rows_ref, pslot, chunk_idx):
            def row_body(r, carry):
                base = r * LSEQ
                accs = [rows_ref[base + k, :] for k in range(NACC)]
                for j in range(NACC, LSEQ):
                    accs[j % NACC] = accs[j % NACC] + rows_ref[base + j, :]
                s = accs[0]
                for a in accs[1:]:
                    s = s + a
                pool_v[pslot, r, :] = s
                return carry

            lax.fori_loop(0, CHUNK, row_body, 0)
            pltpu.async_copy(
                pool_v.at[pslot],
                out_hbm.at[pl.ds(wrow0 + chunk_idx * CHUNK, CHUNK)],
                sem_p,
            )

        pltpu.sync_copy(x_hbm.at[pl.ds(wrow0, SUPER)], idx_a)
        pltpu.async_copy(x_hbm.at[pl.ds(wrow0 + SUPER, SUPER)], idx_b, sem_i)
        fire_gathers(idx_a, 0, rows0, sem0)

        def body(t, carry):
            c0 = 4 * t
            not_last = t < (NSUPER // 2 - 1)

            fire_gathers(idx_a, CHUNK, rows1, sem1)
            drain_rows(rows0, sem0)

            @pl.when(t > 0)
            def _():
                drain_pools(4)

            reduce_store(rows0, 0, c0)
            drain_idx(idx_b)
            fire_gathers(idx_b, 0, rows0, sem0)
            drain_rows(rows1, sem1)
            reduce_store(rows1, 1, c0 + 1)

            @pl.when(not_last)
            def _():
                pltpu.async_copy(
                    x_hbm.at[pl.ds(wrow0 + (2 * t + 2) * SUPER, SUPER)],
                    idx_a, sem_i)

            fire_gathers(idx_b, CHUNK, rows1, sem1)
            drain_rows(rows0, sem0)
            reduce_store(rows0, 2, c0 + 2)

            @pl.when(not_last)
            def _():
                drain_idx(idx_a)

            drain_rows(rows1, sem1)

            @pl.when(not_last)
            def _():
                pltpu.async_copy(
                    x_hbm.at[pl.ds(wrow0 + (2 * t + 3) * SUPER, SUPER)],
                    idx_b, sem_i)

            reduce_store(rows1, 3, c0 + 3)

            @pl.when(not_last)
            def _():
                fire_gathers(idx_a, 0, rows0, sem0)
            return carry

        lax.fori_loop(0, NSUPER // 2, body, 0)
        drain_pools(4)

    return emb_pool(x, emb_table)


def _mlp_body(s_ref, w1_ref, b1_ref, w2_ref, b2_ref, o_ref):
    s = s_ref[...]
    h = jnp.dot(s, w1_ref[...], preferred_element_type=jnp.float32)
    h = jnp.maximum(h + b1_ref[...], 0.0)
    o = jnp.dot(h, w2_ref[...], preferred_element_type=jnp.float32)
    o_ref[...] = o + b2_ref[...]


def _tc_mlp(sums, w1t, b1, w2t, b2):
    return pl.pallas_call(
        _mlp_body,
        out_shape=jax.ShapeDtypeStruct((B, OUT), jnp.float32),
    )(sums, w1t, b1, w2t, b2)


def kernel(x, emb_table, W1, b1, W2, b2):
    sums = _sc_emb_pool_sum(x, emb_table)
    w1t = W1.T / float(LSEQ)
    w2t = W2.T
    return _tc_mlp(sums, w1t, b1.reshape(1, D), w2t, b2.reshape(1, OUT))


# speedup vs baseline: 10.9740x; 1.2342x over previous
import functools

import jax
import jax.numpy as jnp
from jax import lax
from jax.experimental import pallas as pl
from jax.experimental.pallas import tpu as pltpu
from jax.experimental.pallas import tpu_sc as plsc

B = 16384
LSEQ = 200
D = 16
OUT = 2

NC = 2
NS = 16
NW = NC * NS

ROWS_PER_W = B // NW
CHUNK = 16
SUPER = 2 * CHUNK
NSUPER = ROWS_PER_W // SUPER
NACC = 8
ROWS_BUF = CHUNK * LSEQ


def _sc_emb_pool_sum(x, emb_table):
    mesh = plsc.VectorSubcoreMesh(core_axis_name="c", subcore_axis_name="s")

    @functools.partial(
        pl.kernel,
        mesh=mesh,
        out_type=jax.ShapeDtypeStruct((B, D), jnp.float32),
        scratch_types=[
            pltpu.VMEM((SUPER, LSEQ), jnp.int32),
            pltpu.VMEM((SUPER, LSEQ), jnp.int32),
            pltpu.VMEM((ROWS_BUF, D), jnp.float32),
            pltpu.VMEM((ROWS_BUF, D), jnp.float32),
            pltpu.VMEM((4, CHUNK, D), jnp.float32),
            pltpu.SemaphoreType.DMA,
            pltpu.SemaphoreType.DMA,
            pltpu.SemaphoreType.DMA,
            pltpu.SemaphoreType.DMA,
        ],
        compiler_params=pltpu.CompilerParams(use_tc_tiling_on_sc=False),
    )
    def emb_pool(x_hbm, table_hbm, out_hbm, idx_a, idx_b, rows0, rows1,
                 pool_v, sem0, sem1, sem_i, sem_p):
        wid = lax.axis_index("s") * NC + lax.axis_index("c")
        wrow0 = wid * ROWS_PER_W

        def fire_gathers(idx_ref, roff, rows_ref, sem):
            for r in range(CHUNK):
                pltpu.async_copy(
                    table_hbm.at[idx_ref.at[roff + r]],
                    rows_ref.at[pl.ds(r * LSEQ, LSEQ)],
                    sem,
                )

        def drain_rows(rows_ref, sem):
            pltpu.make_async_copy(
                table_hbm.at[pl.ds(0, ROWS_BUF)], rows_ref, sem).wait()

        def drain_idx(idx_ref):
            pltpu.make_async_copy(
                x_hbm.at[pl.ds(0, SUPER)], idx_ref, sem_i).wait()

        def drain_pools(n):
            for _ in range(n):
                pltpu.make_async_copy(
                    pool_v.at[0], out_hbm.at[pl.ds(0, CHUNK)], sem_p).wait()

        def reduce_store(rows_ref, pslot, chunk_idx):
            def row_body(r, carry):
                base = r * LSEQ
                accs = [rows_ref[base + k, :] for k in range(NACC)]
                for j in range(NACC, LSEQ):
                    accs[j % NACC] = accs[j % NACC] + rows_ref[base + j, :]
                s = accs[0]
                for a in accs[1:]:
                    s = s + a
                pool_v[pslot, r, :] = s
                return carry

            lax.fori_loop(0, CHUNK, row_body, 0)
            pltpu.async_copy(
                pool_v.at[pslot],
                out_hbm.at[pl.ds(wrow0 + chunk_idx * CHUNK, CHUNK)],
                sem_p,
            )

        pltpu.sync_copy(x_hbm.at[pl.ds(wrow0, SUPER)], idx_a)
        pltpu.async_copy(x_hbm.at[pl.ds(wrow0 + SUPER, SUPER)], idx_b, sem_i)
        fire_gathers(idx_a, 0, rows0, sem0)

        def body(t, carry):
            c0 = 4 * t
            not_last = t < (NSUPER // 2 - 1)

            fire_gathers(idx_a, CHUNK, rows1, sem1)
            drain_rows(rows0, sem0)

            @pl.when(t > 0)
            def _():
                drain_pools(4)

            reduce_store(rows0, 0, c0)
            drain_idx(idx_b)
            fire_gathers(idx_b, 0, rows0, sem0)
            drain_rows(rows1, sem1)
            reduce_store(rows1, 1, c0 + 1)

            @pl.when(not_last)
            def _():
                pltpu.async_copy(
                    x_hbm.at[pl.ds(wrow0 + (2 * t + 2) * SUPER, SUPER)],
                    idx_a, sem_i)

            fire_gathers(idx_b, CHUNK, rows1, sem1)
            drain_rows(rows0, sem0)
            reduce_store(rows0, 2, c0 + 2)

            @pl.when(not_last)
            def _():
                drain_idx(idx_a)

            drain_rows(rows1, sem1)

            @pl.when(not_last)
            def _():
                pltpu.async_copy(
                    x_hbm.at[pl.ds(wrow0 + (2 * t + 3) * SUPER, SUPER)],
                    idx_b, sem_i)

            reduce_store(rows1, 3, c0 + 3)

            @pl.when(not_last)
            def _():
                fire_gathers(idx_a, 0, rows0, sem0)
            return carry

        lax.fori_loop(0, NSUPER // 2, body, 0)
        drain_pools(4)

    return emb_pool(x, emb_table)


REPACK_BC = 16000
REPACK_BR = REPACK_BC // 8
REPACK_GRID = (1000000 + REPACK_BC - 1) // REPACK_BC


def _repack_body(tt_ref, o_ref):
    t = tt_ref[...].T
    t3 = t.reshape(REPACK_BR, 8, 16)
    o_ref[...] = jnp.concatenate([t3[:, g, :] for g in range(8)], axis=1)


def _tc_repack(table_t):
    return pl.pallas_call(
        _repack_body,
        grid=(REPACK_GRID,),
        in_specs=[pl.BlockSpec((16, REPACK_BC), lambda i: (0, i))],
        out_specs=pl.BlockSpec((REPACK_BR, 128), lambda i: (i, 0)),
        out_shape=jax.ShapeDtypeStruct((125000, 128), jnp.float32),
    )(table_t)


def _mlp_body(s_ref, w1_ref, b1_ref, w2_ref, b2_ref, o_ref):
    s = s_ref[...]
    h = jnp.dot(s, w1_ref[...], preferred_element_type=jnp.float32)
    h = jnp.maximum(h + b1_ref[...], 0.0)
    o = jnp.dot(h, w2_ref[...], preferred_element_type=jnp.float32)
    o_ref[...] = o + b2_ref[...]


def _tc_mlp(sums, w1t, b1, w2t, b2):
    return pl.pallas_call(
        _mlp_body,
        out_shape=jax.ShapeDtypeStruct((B, OUT), jnp.float32),
    )(sums, w1t, b1, w2t, b2)


def kernel(x, emb_table, W1, b1, W2, b2):
    packed = _tc_repack(emb_table.T)
    sums = _sc_emb_pool_sum(x, packed.reshape(1000000, 16))
    w1t = W1.T / float(LSEQ)
    w2t = W2.T
    return _tc_mlp(sums, w1t, b1.reshape(1, D), w2t, b2.reshape(1, OUT))
